# baseline (speedup 1.0000x reference)
import functools

import jax
import jax.numpy as jnp
from jax import lax
from jax.experimental import pallas as pl
from jax.experimental.pallas import tpu as pltpu
from jax.experimental.pallas import tpu_sc as plsc

D_MODEL = 64
SCALE = 8.0
NUM_CORES = 2
NUM_SUBCORES = 16
LANES = 16
NUM_WORKERS = NUM_CORES * NUM_SUBCORES
WINDOW = 512


def _embed_lookup(idx, table):
    batch = idx.shape[0]
    assert batch % (NUM_WORKERS * WINDOW) == 0
    b_per_w = batch // NUM_WORKERS
    n_win = b_per_w // WINDOW

    mesh = plsc.VectorSubcoreMesh(core_axis_name="c", subcore_axis_name="s")

    @functools.partial(
        pl.kernel,
        mesh=mesh,
        compiler_params=pltpu.CompilerParams(use_tc_tiling_on_sc=False),
        out_type=jax.ShapeDtypeStruct((batch, D_MODEL), jnp.float32),
        scratch_types=[
            pltpu.VMEM((WINDOW,), jnp.int32),
            pltpu.VMEM((WINDOW, D_MODEL), jnp.float32),
            pltpu.SemaphoreType.DMA,
        ],
    )
    def k(idx_hbm, table_hbm, out_hbm, idx_v, rows_v, sem):
        wid = lax.axis_index("s") * NUM_CORES + lax.axis_index("c")

        @pl.loop(0, n_win)
        def _(w):
            base = wid * b_per_w + w * WINDOW
            pltpu.sync_copy(idx_hbm.at[pl.ds(base, WINDOW)], idx_v)
            pltpu.async_copy(table_hbm.at[idx_v], rows_v, sem).wait()

            @pl.loop(0, WINDOW)
            def _(r):
                @pl.loop(0, D_MODEL, step=LANES)
                def _(c):
                    slc = (pl.ds(r, 1), pl.ds(c, LANES))
                    rows_v.at[slc][...] = rows_v.at[slc][...] * SCALE

            pltpu.sync_copy(rows_v, out_hbm.at[pl.ds(base, WINDOW)])

    return k(idx, table)


def kernel(x, table):
    rows, cols = x.shape
    idx = x.reshape(rows * cols).astype(jnp.int32)
    out = _embed_lookup(idx, table)
    return out.reshape(rows, cols, D_MODEL)


# speedup vs baseline: 1.0915x; 1.0915x over previous
import functools

import jax
import jax.numpy as jnp
from jax import lax
from jax.experimental import pallas as pl
from jax.experimental.pallas import tpu as pltpu
from jax.experimental.pallas import tpu_sc as plsc

D_MODEL = 64
SCALE = 8.0
NUM_CORES = 2
NUM_SUBCORES = 16
LANES = 16
NUM_WORKERS = NUM_CORES * NUM_SUBCORES
NBUF = 4
WINDOW = 320


def _embed_lookup(idx, table):
    batch = idx.shape[0]
    b_per_w = batch // NUM_WORKERS
    assert batch % NUM_WORKERS == 0 and b_per_w % WINDOW == 0
    n_win = b_per_w // WINDOW
    assert n_win % NBUF == 0

    mesh = plsc.VectorSubcoreMesh(core_axis_name="c", subcore_axis_name="s")

    @functools.partial(
        pl.kernel,
        mesh=mesh,
        compiler_params=pltpu.CompilerParams(use_tc_tiling_on_sc=False),
        out_type=jax.ShapeDtypeStruct((batch, D_MODEL), jnp.float32),
        scratch_types=[
            pltpu.VMEM((b_per_w,), jnp.int32),
        ]
        + [pltpu.VMEM((WINDOW, D_MODEL), jnp.float32)] * NBUF
        + [pltpu.SemaphoreType.DMA] * (2 * NBUF),
    )
    def k(idx_hbm, table_hbm, out_hbm, idx_v, *bufs_and_sems):
        bufs = bufs_and_sems[:NBUF]
        gsem = bufs_and_sems[NBUF : 2 * NBUF]
        ssem = bufs_and_sems[2 * NBUF :]

        wid = lax.axis_index("s") * NUM_CORES + lax.axis_index("c")
        base0 = wid * b_per_w
        pltpu.sync_copy(idx_hbm.at[pl.ds(base0, b_per_w)], idx_v)

        def gather_src(w):
            return table_hbm.at[idx_v.at[pl.ds(w * WINDOW, WINDOW)]]

        def out_dst(w):
            return out_hbm.at[pl.ds(base0 + w * WINDOW, WINDOW)]

        for b in range(NBUF):
            pltpu.async_copy(gather_src(b), bufs[b], gsem[b])

        @pl.loop(0, n_win, step=NBUF)
        def _(w):
            for b in range(NBUF):
                wb = w + b
                pltpu.make_async_copy(gather_src(wb), bufs[b], gsem[b]).wait()

                @pl.loop(0, WINDOW)
                def _(r):
                    for c in range(0, D_MODEL, LANES):
                        slc = (pl.ds(r, 1), pl.ds(c, LANES))
                        bufs[b].at[slc][...] = bufs[b].at[slc][...] * SCALE

                pltpu.async_copy(bufs[b], out_dst(wb), ssem[b])

            for b in range(NBUF):
                wb = w + b
                pltpu.make_async_copy(bufs[b], out_dst(wb), ssem[b]).wait()

                @pl.when(wb + NBUF < n_win)
                def _():
                    pltpu.async_copy(gather_src(wb + NBUF), bufs[b], gsem[b])

    return k(idx, table)


def kernel(x, table):
    rows, cols = x.shape
    idx = x.reshape(rows * cols).astype(jnp.int32)
    out = _embed_lookup(idx, table)
    return out.reshape(rows, cols, D_MODEL)
